# baseline (speedup 1.0000x reference)
import functools

import jax
import jax.numpy as jnp
from jax import lax
from jax.experimental import pallas as pl
from jax.experimental.pallas import tpu as pltpu
from jax.experimental.pallas import tpu_sc as plsc

NC = 2
NS = 16
NW = NC * NS
LANES = 16


@functools.lru_cache(maxsize=None)
def _build(B, S, D):
    s_per_w = S // NW
    T = 16
    if s_per_w % T:
        T = s_per_w
    NT = s_per_w // T
    TW = T * D
    NV = TW // LANES

    mesh = plsc.VectorSubcoreMesh(core_axis_name="c", subcore_axis_name="s")

    @functools.partial(
        pl.kernel,
        mesh=mesh,
        out_type=jax.ShapeDtypeStruct((B * S * D,), jnp.float32),
        scratch_types=[
            pltpu.VMEM((B, TW), jnp.float32),
            pltpu.VMEM((TW,), jnp.float32),
            pltpu.SemaphoreType.DMA,
            pltpu.SemaphoreType.DMA,
        ],
    )
    def k(x_hbm, emb_hbm, out_hbm, xbuf, ebuf, sem_i, sem_o):
        wid = lax.axis_index("s") * NC + lax.axis_index("c")
        s0 = wid * s_per_w

        @pl.loop(0, NT)
        def _step(t):
            e_off = (s0 + t * T) * D
            cp_e = pltpu.async_copy(emb_hbm.at[pl.ds(e_off, TW)], ebuf, sem_i)
            cps = []
            for b in range(B):
                cps.append(pltpu.async_copy(
                    x_hbm.at[pl.ds(b * S * D + e_off, TW)], xbuf.at[b], sem_i))
            cp_e.wait()
            for cp in cps:
                cp.wait()
            for b in range(B):
                @pl.loop(0, NV)
                def _add(i, b=b):
                    o = pl.ds(i * LANES, LANES)
                    xbuf[b, o] = xbuf[b, o] + ebuf[o]
            outs = []
            for b in range(B):
                outs.append(pltpu.async_copy(
                    xbuf.at[b], out_hbm.at[pl.ds(b * S * D + e_off, TW)], sem_o))
            for cp in outs:
                cp.wait()

    return k


def kernel(x, emb_table):
    B, S, D = x.shape
    out = _build(B, S, D)(x.reshape(-1), emb_table.reshape(-1))
    return out.reshape(B, S, D)


# speedup vs baseline: 1.2063x; 1.2063x over previous
import functools

import jax
import jax.numpy as jnp
from jax import lax
from jax.experimental import pallas as pl
from jax.experimental.pallas import tpu as pltpu
from jax.experimental.pallas import tpu_sc as plsc

NC = 2
NS = 16
NW = NC * NS
LANES = 16


@functools.lru_cache(maxsize=None)
def _build(B, S, D):
    s_per_w = S // NW
    T = 8
    if s_per_w % T:
        T = s_per_w
    NT = s_per_w // T
    TW = T * D
    NV = TW // LANES

    mesh = plsc.VectorSubcoreMesh(core_axis_name="c", subcore_axis_name="s")

    @functools.partial(
        pl.kernel,
        mesh=mesh,
        out_type=jax.ShapeDtypeStruct((B * S * D,), jnp.float32),
        scratch_types=[
            pltpu.VMEM((2 * B, TW), jnp.float32),
            pltpu.VMEM((2, TW), jnp.float32),
            pltpu.SemaphoreType.DMA((2 * B,)),
            pltpu.SemaphoreType.DMA((2,)),
            pltpu.SemaphoreType.DMA((2 * B,)),
        ],
    )
    def k(x_hbm, emb_hbm, out_hbm, xbuf, ebuf, sem_x, sem_e, sem_o):
        wid = lax.axis_index("s") * NC + lax.axis_index("c")
        s0 = wid * s_per_w

        def start_e(t):
            return pltpu.async_copy(
                emb_hbm.at[pl.ds((s0 + t * T) * D, TW)],
                ebuf.at[t % 2], sem_e.at[t % 2])

        def start_x(t, b):
            slot = (t % 2) * B + b
            return pltpu.async_copy(
                x_hbm.at[pl.ds(b * S * D + (s0 + t * T) * D, TW)],
                xbuf.at[slot], sem_x.at[slot])

        def start_o(t, b):
            slot = (t % 2) * B + b
            return pltpu.async_copy(
                xbuf.at[slot],
                out_hbm.at[pl.ds(b * S * D + (s0 + t * T) * D, TW)],
                sem_o.at[slot])

        e_cp, x_cp, o_cp = {}, {}, {}
        e_cp[0] = start_e(0)
        if NT > 1:
            e_cp[1] = start_e(1)
        for b in range(B):
            x_cp[(0, b)] = start_x(0, b)
        if NT > 1:
            for b in range(B):
                x_cp[(1, b)] = start_x(1, b)

        for t in range(NT):
            e_cp[t].wait()
            for b in range(B):
                slot = (t % 2) * B + b
                if t >= 1 and t + 1 < NT:
                    o_cp[(t - 1, b)].wait()
                    x_cp[(t + 1, b)] = start_x(t + 1, b)
                elif t >= 1:
                    o_cp[(t - 1, b)].wait()
                x_cp[(t, b)].wait()
                tp = t % 2

                @pl.loop(0, NV, unroll=8)
                def _add(i, slot=slot, tp=tp):
                    o = pl.ds(i * LANES, LANES)
                    plsc.addupdate(xbuf.at[slot, o], ebuf[tp, o])

                o_cp[(t, b)] = start_o(t, b)
            if t + 2 < NT:
                e_cp[t + 2] = start_e(t + 2)
        for b in range(B):
            o_cp[(NT - 1, b)].wait()

    return k


def kernel(x, emb_table):
    B, S, D = x.shape
    out = _build(B, S, D)(x.reshape(-1), emb_table.reshape(-1))
    return out.reshape(B, S, D)


# speedup vs baseline: 5.3112x; 4.4029x over previous
import functools

import jax
import jax.numpy as jnp
from jax import lax
from jax.experimental import pallas as pl
from jax.experimental.pallas import tpu as pltpu
from jax.experimental.pallas import tpu_sc as plsc

NC = 2
NS = 16
NW = NC * NS
LANES = 16


@functools.lru_cache(maxsize=None)
def _build(B, S, D):
    s_per_w = S // NW
    T = 8
    if s_per_w % T:
        T = s_per_w
    NT = s_per_w // T
    NVD = D // LANES

    mesh = plsc.VectorSubcoreMesh(core_axis_name="c", subcore_axis_name="s")

    @functools.partial(
        pl.kernel,
        mesh=mesh,
        out_type=jax.ShapeDtypeStruct((B, S, D), jnp.float32),
        scratch_types=[
            pltpu.VMEM((3, B, T, D), jnp.float32),
            pltpu.VMEM((2, T, D), jnp.float32),
            pltpu.SemaphoreType.DMA((3,)),
            pltpu.SemaphoreType.DMA((2,)),
            pltpu.SemaphoreType.DMA((3,)),
        ],
    )
    def k(x_hbm, emb_hbm, out_hbm, xbuf, ebuf, sem_x, sem_e, sem_o):
        wid = lax.axis_index("s") * NC + lax.axis_index("c")
        s0 = wid * s_per_w

        def start_e(g):
            return pltpu.async_copy(
                emb_hbm.at[pl.ds(s0 + g * T, T)],
                ebuf.at[g % 2], sem_e.at[g % 2])

        def start_x(g):
            return pltpu.async_copy(
                x_hbm.at[:, pl.ds(s0 + g * T, T)],
                xbuf.at[g % 3], sem_x.at[g % 3])

        def start_o(g):
            return pltpu.async_copy(
                xbuf.at[g % 3],
                out_hbm.at[:, pl.ds(s0 + g * T, T)],
                sem_o.at[g % 3])

        e_cp, x_cp, o_cp = {}, {}, {}
        for g in range(min(2, NT)):
            e_cp[g] = start_e(g)
        for g in range(min(3, NT)):
            x_cp[g] = start_x(g)

        for g in range(NT):
            e_cp[g].wait()
            if g >= 1:
                o_cp[g - 1].wait()
                if g + 2 < NT:
                    x_cp[g + 2] = start_x(g + 2)
            x_cp[g].wait()
            ep = g % 2
            sx = g % 3
            @pl.loop(0, T)
            def _row(r, sx=sx, ep=ep):
                @pl.loop(0, NVD, unroll=8)
                def _add(i):
                    o = pl.ds(i * LANES, LANES)
                    v = ebuf[ep, r, o]
                    for b in range(B):
                        plsc.addupdate(xbuf.at[sx, b, r, o], v)

            o_cp[g] = start_o(g)
            if g + 2 < NT:
                e_cp[g + 2] = start_e(g + 2)
        for g in range(max(0, NT - 1), NT):
            o_cp[g].wait()

    return k


def kernel(x, emb_table):
    B, S, D = x.shape
    return _build(B, S, D)(x, emb_table)
